# baseline (speedup 1.0000x reference)
import functools

import jax
import jax.numpy as jnp
from jax.experimental import pallas as pl
from jax.experimental.pallas import tpu as pltpu

_KSIZE = 7
_PAD = _KSIZE // 2
_LANES = 128


def _build_tap_matrices(weight, W, F, ns, pt):
    q = jnp.arange(_LANES)
    w = jnp.arange(_LANES)
    b_src, c_src = q // W, q % W
    b_out, c_out = w // W, w % W
    wf = weight.reshape(2, _KSIZE, _KSIZE).astype(jnp.float32)
    per_d = []
    for d in range(ns):
        acc = jnp.zeros((2, _LANES, _LANES), jnp.float32)
        for i in range(_KSIZE):
            dr = i - _PAD
            delta = (b_out + dr) // F
            bs = (b_out + dr) % F
            row_ok = (delta == (d - pt)) & (b_src[:, None] == bs[None, :])
            for j in range(_KSIZE):
                dc = j - _PAD
                m = row_ok & (c_src[:, None] == (c_out + dc)[None, :])
                acc = acc + wf[:, i, j][:, None, None] * m[None].astype(jnp.float32)
        per_d.append(acc)
    return jnp.stack(per_d, axis=1)


def _fused_body(*refs, c_total, hf, ns, n_split, out_dtype):
    x_refs = refs[:n_split]
    t_ref, o_ref, pad_ref = refs[n_split:]
    pt = (ns - 1) // 2
    psums, pmaxs = [], []
    for xr in x_refs:
        xq = xr[0, 0]
        psums.append(jnp.sum(xq, axis=0))
        pmaxs.append(jnp.max(xq, axis=0))
    mean = functools.reduce(jnp.add, psums) * (1.0 / float(c_total))
    mx = functools.reduce(jnp.maximum, pmaxs)

    zrow = jnp.zeros((pt, _LANES), jnp.float32)
    for ci in range(2):
        pad_ref[ci, :pt, :] = zrow
        pad_ref[ci, pt + hf:, :] = zrow
    pad_ref[0, pt:pt + hf, :] = mean
    pad_ref[1, pt:pt + hf, :] = mx

    acc = jnp.zeros((hf, _LANES), jnp.float32)
    for ci in range(2):
        for d in range(ns):
            r = pad_ref[ci, d:d + hf, :]
            acc = acc + jnp.dot(r, t_ref[ci, d],
                                preferred_element_type=jnp.float32)
    o_ref[0, 0] = jax.nn.sigmoid(acc).astype(out_dtype)


def kernel(x, weight):
    N, C, H, W = x.shape
    assert weight.shape == (1, 2, _KSIZE, _KSIZE)
    assert _LANES % W == 0, "W must divide 128"
    F = _LANES // W
    assert H % F == 0, "H must be divisible by the fold factor"
    hf = H // F
    assert hf % 8 == 0, "folded height must be sublane-aligned"

    pt = (_PAD + F - 1) // F
    pb = (F - 1 + _PAD) // F
    assert pt == pb
    ns = pt + pb + 1

    n_split = 4 if C % 4 == 0 else 1
    t_mats = _build_tap_matrices(weight, W, F, ns, pt)
    xs = x.reshape(N, n_split, C // n_split, hf, _LANES)

    body = functools.partial(_fused_body, c_total=C, hf=hf, ns=ns,
                             n_split=n_split, out_dtype=x.dtype)
    block_bytes = C * hf * _LANES * jnp.dtype(x.dtype).itemsize
    vmem_limit = int(min(2 * block_bytes + (8 << 20), 56 << 20))

    def _xspec(q):
        return pl.BlockSpec((1, 1, C // n_split, hf, _LANES),
                            lambda n: (n, q, 0, 0, 0))

    out = pl.pallas_call(
        body,
        out_shape=jax.ShapeDtypeStruct((N, 1, hf, _LANES), x.dtype),
        grid=(N,),
        in_specs=[_xspec(q) for q in range(n_split)] + [
            pl.BlockSpec((2, ns, _LANES, _LANES), lambda n: (0, 0, 0, 0)),
        ],
        out_specs=pl.BlockSpec((1, 1, hf, _LANES), lambda n: (n, 0, 0, 0)),
        scratch_shapes=[pltpu.VMEM((2, hf + ns - 1, _LANES), jnp.float32)],
        compiler_params=pltpu.CompilerParams(
            dimension_semantics=("arbitrary",),
            vmem_limit_bytes=vmem_limit),
    )(*([xs] * n_split), t_mats)
    return out.reshape(N, 1, H, W)


# speedup vs baseline: 1.4689x; 1.4689x over previous
import functools

import jax
import jax.numpy as jnp
from jax.experimental import pallas as pl
from jax.experimental.pallas import tpu as pltpu

_KSIZE = 7
_PAD = _KSIZE // 2


def _build_tap_matrices(weight, W):
    d = jnp.arange(W)[:, None] - jnp.arange(W)[None, :]
    masks = jnp.stack([(d == (kw - _PAD)) for kw in range(_KSIZE)])
    wf = weight.reshape(2, _KSIZE, _KSIZE).astype(jnp.float32)
    return jnp.einsum("ckj,jqw->ckqw", wf, masks.astype(jnp.float32))


def _fused_body(x_ref, t_ref, o_ref, pad_ref, *, c_total, h, out_dtype):
    x = x_ref[0]
    mean = jnp.sum(x, axis=0) * (1.0 / float(c_total))
    mx = jnp.max(x, axis=0)

    zrow = jnp.zeros_like(pad_ref[0, :_PAD, :])
    for ci in range(2):
        pad_ref[ci, :_PAD, :] = zrow
        pad_ref[ci, _PAD + h:, :] = zrow
    pad_ref[0, _PAD:_PAD + h, :] = mean
    pad_ref[1, _PAD:_PAD + h, :] = mx

    acc = jnp.zeros_like(mean)
    for ci in range(2):
        for kh in range(_KSIZE):
            r = pad_ref[ci, kh:kh + h, :]
            acc = acc + jnp.dot(r, t_ref[ci, kh],
                                preferred_element_type=jnp.float32)
    o_ref[0, 0] = jax.nn.sigmoid(acc).astype(out_dtype)


def kernel(x, weight):
    N, C, H, W = x.shape
    assert weight.shape == (1, 2, _KSIZE, _KSIZE)

    t_mats = _build_tap_matrices(weight, W)
    body = functools.partial(_fused_body, c_total=C, h=H, out_dtype=x.dtype)

    lanes = -(-W // 128) * 128
    block_bytes = C * H * lanes * jnp.dtype(x.dtype).itemsize
    vmem_limit = int(min(2 * block_bytes + (8 << 20), 56 << 20))

    return pl.pallas_call(
        body,
        out_shape=jax.ShapeDtypeStruct((N, 1, H, W), x.dtype),
        grid=(N,),
        in_specs=[
            pl.BlockSpec((1, C, H, W), lambda n: (n, 0, 0, 0)),
            pl.BlockSpec((2, _KSIZE, W, W), lambda n: (0, 0, 0, 0)),
        ],
        out_specs=pl.BlockSpec((1, 1, H, W), lambda n: (n, 0, 0, 0)),
        scratch_shapes=[pltpu.VMEM((2, H + 2 * _PAD, W), jnp.float32)],
        compiler_params=pltpu.CompilerParams(
            dimension_semantics=("arbitrary",),
            vmem_limit_bytes=vmem_limit),
    )(x, t_mats)


# speedup vs baseline: 2.5837x; 1.7589x over previous
import functools

import jax
import jax.numpy as jnp
from jax.experimental import pallas as pl
from jax.experimental.pallas import tpu as pltpu

_KSIZE = 7
_PAD = _KSIZE // 2
_LANES = 128


def _build_tap_matrices(weight, W, F, ns, pt):
    q = jnp.arange(_LANES)
    w = jnp.arange(_LANES)
    b_src, c_src = q // W, q % W
    b_out, c_out = w // W, w % W
    wf = weight.reshape(2, _KSIZE, _KSIZE).astype(jnp.float32)
    per_d = []
    for d in range(ns):
        acc = jnp.zeros((2, _LANES, _LANES), jnp.float32)
        for i in range(_KSIZE):
            dr = i - _PAD
            delta = (b_out + dr) // F
            bs = (b_out + dr) % F
            row_ok = (delta == (d - pt)) & (b_src[:, None] == bs[None, :])
            for j in range(_KSIZE):
                dc = j - _PAD
                m = row_ok & (c_src[:, None] == (c_out + dc)[None, :])
                acc = acc + wf[:, i, j][:, None, None] * m[None].astype(jnp.float32)
        per_d.append(acc)
    return jnp.stack(per_d, axis=1)


def _fused_body(x_hbm, t_ref, o_ref, xbuf, sems, pad_ref, *,
                c_total, hf, ns, n_stream, out_dtype):
    pt = (ns - 1) // 2
    n = pl.program_id(0)
    num = pl.num_programs(0)
    cc = c_total // n_stream
    cur = jax.lax.rem(n, 2)
    nxt = jax.lax.rem(n + 1, 2)

    def _issue(img, half):
        for s in range(n_stream):
            pltpu.make_async_copy(
                x_hbm.at[img, pl.ds(s * cc, cc)],
                xbuf.at[half, s],
                sems.at[half, s],
            ).start()

    @pl.when(n == 0)
    def _():
        _issue(0, 0)

    @pl.when(n + 1 < num)
    def _():
        _issue(n + 1, nxt)

    for s in range(n_stream):
        pltpu.make_async_copy(
            x_hbm.at[n, pl.ds(s * cc, cc)],
            xbuf.at[cur, s],
            sems.at[cur, s],
        ).wait()

    x = xbuf[cur].reshape(c_total, hf, _LANES)
    mean = jnp.sum(x, axis=0) * (1.0 / float(c_total))
    mx = jnp.max(x, axis=0)

    zrow = jnp.zeros((pt, _LANES), jnp.float32)
    for ci in range(2):
        pad_ref[ci, :pt, :] = zrow
        pad_ref[ci, pt + hf:, :] = zrow
    pad_ref[0, pt:pt + hf, :] = mean
    pad_ref[1, pt:pt + hf, :] = mx

    acc = jnp.zeros((hf, _LANES), jnp.float32)
    for ci in range(2):
        for d in range(ns):
            r = pad_ref[ci, d:d + hf, :]
            acc = acc + jnp.dot(r, t_ref[ci, d],
                                preferred_element_type=jnp.float32)
    o_ref[0, 0] = jax.nn.sigmoid(acc).astype(out_dtype)


def kernel(x, weight):
    N, C, H, W = x.shape
    assert weight.shape == (1, 2, _KSIZE, _KSIZE)
    assert _LANES % W == 0, "W must divide 128"
    F = _LANES // W
    assert H % F == 0, "H must be divisible by the fold factor"
    hf = H // F
    assert hf % 8 == 0, "folded height must be sublane-aligned"

    pt = (_PAD + F - 1) // F
    pb = (F - 1 + _PAD) // F
    assert pt == pb
    ns = pt + pb + 1

    n_stream = 4 if C % 4 == 0 else 1
    t_mats = _build_tap_matrices(weight, W, F, ns, pt)
    xf = x.reshape(N, C, hf, _LANES)

    body = functools.partial(_fused_body, c_total=C, hf=hf, ns=ns,
                             n_stream=n_stream, out_dtype=x.dtype)
    block_bytes = C * hf * _LANES * jnp.dtype(x.dtype).itemsize
    vmem_limit = int(min(2 * block_bytes + (8 << 20), 56 << 20))

    out = pl.pallas_call(
        body,
        out_shape=jax.ShapeDtypeStruct((N, 1, hf, _LANES), x.dtype),
        grid=(N,),
        in_specs=[
            pl.BlockSpec(memory_space=pl.ANY),
            pl.BlockSpec((2, ns, _LANES, _LANES), lambda n: (0, 0, 0, 0)),
        ],
        out_specs=pl.BlockSpec((1, 1, hf, _LANES), lambda n: (n, 0, 0, 0)),
        scratch_shapes=[
            pltpu.VMEM((2, n_stream, C // n_stream, hf, _LANES), jnp.float32),
            pltpu.SemaphoreType.DMA((2, n_stream)),
            pltpu.VMEM((2, hf + ns - 1, _LANES), jnp.float32),
        ],
        compiler_params=pltpu.CompilerParams(
            dimension_semantics=("arbitrary",),
            vmem_limit_bytes=vmem_limit),
    )(xf, t_mats)
    return out.reshape(N, 1, H, W)


# speedup vs baseline: 2.5893x; 1.0022x over previous
import functools

import jax
import jax.numpy as jnp
from jax.experimental import pallas as pl
from jax.experimental.pallas import tpu as pltpu

_KSIZE = 7
_PAD = _KSIZE // 2
_LANES = 128


def _build_tap_matrices(weight, W, F, ns, pt):
    q = jnp.arange(_LANES)
    w = jnp.arange(_LANES)
    b_src, c_src = q // W, q % W
    b_out, c_out = w // W, w % W
    wf = weight.reshape(2, _KSIZE, _KSIZE).astype(jnp.float32)
    per_d = []
    for d in range(ns):
        acc = jnp.zeros((2, _LANES, _LANES), jnp.float32)
        for i in range(_KSIZE):
            dr = i - _PAD
            delta = (b_out + dr) // F
            bs = (b_out + dr) % F
            row_ok = (delta == (d - pt)) & (b_src[:, None] == bs[None, :])
            for j in range(_KSIZE):
                dc = j - _PAD
                m = row_ok & (c_src[:, None] == (c_out + dc)[None, :])
                acc = acc + wf[:, i, j][:, None, None] * m[None].astype(jnp.float32)
        per_d.append(acc)
    return jnp.stack(per_d, axis=1)


def _fused_body(x_ref, t_ref, o_ref, pad_ref, *, c_total, hf, ns, out_dtype):
    pt = (ns - 1) // 2
    x = x_ref[0]
    mean = jnp.sum(x, axis=0) * (1.0 / float(c_total))
    mx = jnp.max(x, axis=0)

    zrow = jnp.zeros((pt, _LANES), jnp.float32)
    for ci in range(2):
        pad_ref[ci, :pt, :] = zrow
        pad_ref[ci, pt + hf:, :] = zrow
    pad_ref[0, pt:pt + hf, :] = mean
    pad_ref[1, pt:pt + hf, :] = mx

    acc = jnp.zeros((hf, _LANES), jnp.float32)
    for ci in range(2):
        for d in range(ns):
            r = pad_ref[ci, d:d + hf, :]
            acc = acc + jnp.dot(r, t_ref[ci, d],
                                preferred_element_type=jnp.float32)
    o_ref[0, 0] = jax.nn.sigmoid(acc).astype(out_dtype)


def kernel(x, weight):
    N, C, H, W = x.shape
    assert weight.shape == (1, 2, _KSIZE, _KSIZE)
    assert _LANES % W == 0, "W must divide 128"
    F = _LANES // W
    assert H % F == 0, "H must be divisible by the fold factor"
    hf = H // F
    assert hf % 8 == 0, "folded height must be sublane-aligned"

    pt = (_PAD + F - 1) // F
    pb = (F - 1 + _PAD) // F
    assert pt == pb
    ns = pt + pb + 1

    t_mats = _build_tap_matrices(weight, W, F, ns, pt)
    xf = x.reshape(N, C, hf, _LANES)

    body = functools.partial(_fused_body, c_total=C, hf=hf, ns=ns,
                             out_dtype=x.dtype)
    block_bytes = C * hf * _LANES * jnp.dtype(x.dtype).itemsize
    vmem_limit = int(min(2 * block_bytes + (8 << 20), 56 << 20))

    out = pl.pallas_call(
        body,
        out_shape=jax.ShapeDtypeStruct((N, 1, hf, _LANES), x.dtype),
        grid=(N,),
        in_specs=[
            pl.BlockSpec((1, C, hf, _LANES), lambda n: (n, 0, 0, 0)),
            pl.BlockSpec((2, ns, _LANES, _LANES), lambda n: (0, 0, 0, 0)),
        ],
        out_specs=pl.BlockSpec((1, 1, hf, _LANES), lambda n: (n, 0, 0, 0)),
        scratch_shapes=[pltpu.VMEM((2, hf + ns - 1, _LANES), jnp.float32)],
        compiler_params=pltpu.CompilerParams(
            dimension_semantics=("arbitrary",),
            vmem_limit_bytes=vmem_limit),
    )(xf, t_mats)
    return out.reshape(N, 1, H, W)
